# Initial kernel scaffold; baseline (speedup 1.0000x reference)
import jax
import jax.numpy as jnp
from jax.experimental import pallas as pl


def kernel(x, edge_index, batch, W1a, b1a, W1b, b1b, W2a, b2a, W2b, b2b, W3a, b3a, W3b, b3b, lin_W, lin_b):
    raise NotImplementedError("write your pallas kernel here")


# speedup vs baseline: 3.6702x; 3.6702x over previous
import functools

import jax
import jax.numpy as jnp
from jax import lax
from jax.experimental import pallas as pl
from jax.experimental.pallas import tpu as pltpu
from jax.experimental.pallas import tpu_sc as plsc

_N, _D, _H, _C, _E, _G = 10000, 128, 128, 10, 320000, 128
_NC, _NS = 2, 16
_NP = 10240
_HALF = _NP // _NC
_GARB = _HALF
_CH = 112
_EPT = 20048
_NCHUNK = _EPT // _CH
_RPT = _HALF // _NS
_SR = 160
_DPAD = _N + 8


def _segsum_body(h_hbm, src_hbm, dst_hbm, zrows_hbm, out_hbm,
                 sidx, didx, rows, stg, acc, sem):
  cid = lax.axis_index("c")
  sid = lax.axis_index("s")
  base = cid * _HALF

  pltpu.sync_copy(zrows_hbm, stg)
  def zinit(j, carry):
    pltpu.sync_copy(stg, acc.at[pl.ds(sid * _RPT + j * _SR, _SR)])
    return carry
  lax.fori_loop(0, _RPT // _SR, zinit, 0)

  pltpu.sync_copy(src_hbm.at[sid], sidx)
  pltpu.sync_copy(dst_hbm.at[sid], didx)

  def dloc(r, carry):
    for c in range(_CH // 16):
      v = didx[r, pl.ds(c * 16, 16)] - base
      bad = (v < 0) | (v >= _HALF)
      didx[r, pl.ds(c * 16, 16)] = jnp.where(bad, _GARB, v)
    return carry
  lax.fori_loop(0, _NCHUNK, dloc, 0)
  plsc.subcore_barrier()

  def chunk(cc, carry):
    pltpu.async_copy(h_hbm.at[sidx.at[cc]], rows, sem).wait()
    pltpu.sync_copy(rows, acc.at[didx.at[cc]], add=True)
    return carry
  lax.fori_loop(0, _NCHUNK, chunk, 0)

  plsc.subcore_barrier()

  def outb(j, carry):
    r0 = sid * _RPT + j * _SR
    pltpu.sync_copy(acc.at[pl.ds(r0, _SR)], stg)
    pltpu.sync_copy(stg, out_hbm.at[cid, pl.ds(r0, _SR)])
    return carry
  lax.fori_loop(0, _RPT // _SR, outb, 0)


@functools.cache
def _get_segsum():
  mesh = plsc.VectorSubcoreMesh(
      core_axis_name="c", subcore_axis_name="s",
      num_cores=_NC, num_subcores=_NS)
  return pl.kernel(
      _segsum_body,
      out_type=jax.ShapeDtypeStruct((_NC, _HALF, _D), jnp.float32),
      mesh=mesh,
      scratch_types=[
          pltpu.VMEM((_NCHUNK, _CH), jnp.int32),
          pltpu.VMEM((_NCHUNK, _CH), jnp.int32),
          pltpu.VMEM((_CH, _D), jnp.float32),
          pltpu.VMEM((_SR, _D), jnp.float32),
          pltpu.VMEM_SHARED((_HALF + 8, _D), jnp.float32),
          pltpu.SemaphoreType.DMA,
      ],
  )


def _segsum(h, src, dst, zrows):
  return _get_segsum()(h, src, dst, zrows)


_BLK = 640
_NBLK = _NP // _BLK
_BPH = _HALF // _BLK


def _mlp_body(relu_out, xr, pr, War, bar, Wbr, bbr, outr):
  t = xr[...] + pr[0]
  u = jnp.maximum(
      jnp.dot(t, War[...], preferred_element_type=jnp.float32) + bar[...], 0.0)
  v = jnp.dot(u, Wbr[...], preferred_element_type=jnp.float32) + bbr[...]
  outr[...] = jnp.maximum(v, 0.0) if relu_out else v


def _mlp(x, parts, Wa, ba, Wb, bb, relu_out):
  return pl.pallas_call(
      functools.partial(_mlp_body, relu_out),
      grid=(_NBLK,),
      in_specs=[
          pl.BlockSpec((_BLK, _D), lambda i: (i, 0)),
          pl.BlockSpec((1, _BLK, _D), lambda i: (i // _BPH, i % _BPH, 0)),
          pl.BlockSpec((_D, _H), lambda i: (0, 0)),
          pl.BlockSpec((1, _H), lambda i: (0, 0)),
          pl.BlockSpec((_H, _H), lambda i: (0, 0)),
          pl.BlockSpec((1, _H), lambda i: (0, 0)),
      ],
      out_specs=pl.BlockSpec((_BLK, _H), lambda i: (i, 0)),
      out_shape=jax.ShapeDtypeStruct((_NP, _H), jnp.float32),
  )(x, parts, Wa, ba.reshape(1, _H), Wb, bb.reshape(1, _H))


def _final_body(xr, pr, War, bar, Wbr, bbr, batchr, linWr, linbr,
                outr, midr, sums, cnts):
  i = pl.program_id(0)

  @pl.when(i == 0)
  def _():
    sums[...] = jnp.zeros_like(sums)
    cnts[...] = jnp.zeros_like(cnts)

  t = xr[...] + pr[0]
  u = jnp.maximum(
      jnp.dot(t, War[...], preferred_element_type=jnp.float32) + bar[...], 0.0)
  v = jnp.dot(u, Wbr[...], preferred_element_type=jnp.float32) + bbr[...]
  onehot = (batchr[...] == lax.broadcasted_iota(jnp.int32, (_BLK, _G), 1)
            ).astype(jnp.float32)
  sums[...] += lax.dot_general(onehot, v, (((0,), (0,)), ((), ())),
                               preferred_element_type=jnp.float32)
  cnts[...] += lax.dot_general(onehot, jnp.ones((_BLK, 1), jnp.float32),
                               (((0,), (0,)), ((), ())),
                               preferred_element_type=jnp.float32)

  @pl.when(i == _NBLK - 1)
  def _():
    pooled = sums[...] / jnp.maximum(cnts[...], 1.0)
    midr[...] = pooled
    outr[...] = (jnp.dot(pooled, linWr[...], preferred_element_type=jnp.float32)
                 + linbr[...])


def _final(x, parts, Wa, ba, Wb, bb, batch2d, lin_W, lin_b):
  return pl.pallas_call(
      _final_body,
      grid=(_NBLK,),
      in_specs=[
          pl.BlockSpec((_BLK, _D), lambda i: (i, 0)),
          pl.BlockSpec((1, _BLK, _D), lambda i: (i // _BPH, i % _BPH, 0)),
          pl.BlockSpec((_D, _H), lambda i: (0, 0)),
          pl.BlockSpec((1, _H), lambda i: (0, 0)),
          pl.BlockSpec((_H, _H), lambda i: (0, 0)),
          pl.BlockSpec((1, _H), lambda i: (0, 0)),
          pl.BlockSpec((_BLK, 1), lambda i: (i, 0)),
          pl.BlockSpec((_H, _C), lambda i: (0, 0)),
          pl.BlockSpec((1, _C), lambda i: (0, 0)),
      ],
      out_specs=[
          pl.BlockSpec((_G, _C), lambda i: (0, 0)),
          pl.BlockSpec((_G, _H), lambda i: (0, 0)),
      ],
      out_shape=[
          jax.ShapeDtypeStruct((_G, _C), jnp.float32),
          jax.ShapeDtypeStruct((_G, _H), jnp.float32),
      ],
      scratch_shapes=[
          pltpu.VMEM((_G, _H), jnp.float32),
          pltpu.VMEM((_G, 1), jnp.float32),
      ],
  )(x, parts, Wa, ba.reshape(1, _H), Wb, bb.reshape(1, _H),
    batch2d, lin_W, lin_b.reshape(1, _C))


def kernel(x, edge_index, batch, W1a, b1a, W1b, b1b, W2a, b2a, W2b, b2b,
           W3a, b3a, W3b, b3b, lin_W, lin_b):
  pad_e = _NS * _EPT - _E
  src = jnp.pad(edge_index[0].reshape(_NS, _E // _NS),
                ((0, 0), (0, pad_e // _NS))).reshape(_NS, _NCHUNK, _CH)
  dst = jnp.pad(edge_index[1].reshape(_NS, _E // _NS),
                ((0, 0), (0, pad_e // _NS)),
                constant_values=_DPAD).reshape(_NS, _NCHUNK, _CH)
  zrows = jnp.zeros((_SR, _D), jnp.float32)
  xp = jnp.pad(x, ((0, _NP - _N), (0, 0)))
  batch2d = jnp.pad(batch, (0, _NP - _N), constant_values=_G).reshape(_NP, 1)

  p = _segsum(xp, src, dst, zrows)
  h = _mlp(xp, p, W1a, b1a, W1b, b1b, relu_out=True)
  p = _segsum(h, src, dst, zrows)
  h = _mlp(h, p, W2a, b2a, W2b, b2b, relu_out=True)
  p = _segsum(h, src, dst, zrows)
  out, x_mid = _final(h, p, W3a, b3a, W3b, b3b, batch2d, lin_W, lin_b)
  return (out, x_mid)


# speedup vs baseline: 3.9684x; 1.0812x over previous
import functools

import jax
import jax.numpy as jnp
from jax import lax
from jax.experimental import pallas as pl
from jax.experimental.pallas import tpu as pltpu
from jax.experimental.pallas import tpu_sc as plsc

_N, _D, _H, _C, _E, _G = 10000, 128, 128, 10, 320000, 128
_NC, _NS = 2, 16
_NP = 10240
_HALF = _NP // _NC
_GARB = _HALF
_CH = 128
_EPT = 20096
_NCHUNK = _EPT // _CH
_RPT = _HALF // _NS
_SR = 64
_DPAD = _N + 8


def _segsum_body(h_hbm, src_hbm, dst_hbm, zrows_hbm, out_hbm,
                 sidx, didx, rows, stg, acc, sems):
  cid = lax.axis_index("c")
  sid = lax.axis_index("s")
  base = cid * _HALF

  pltpu.sync_copy(zrows_hbm, stg)
  def zinit(j, carry):
    pltpu.sync_copy(stg, acc.at[pl.ds(sid * _RPT + j * _SR, _SR)])
    return carry
  lax.fori_loop(0, _RPT // _SR, zinit, 0)

  pltpu.sync_copy(src_hbm.at[sid], sidx)
  pltpu.sync_copy(dst_hbm.at[sid], didx)

  def dloc(r, carry):
    for c in range(_CH // 16):
      v = didx[r, pl.ds(c * 16, 16)] - base
      bad = (v < 0) | (v >= _HALF)
      didx[r, pl.ds(c * 16, 16)] = jnp.where(bad, _GARB, v)
    return carry
  lax.fori_loop(0, _NCHUNK, dloc, 0)
  plsc.subcore_barrier()

  def chunk(cc, carry):
    b = lax.rem(cc, 2)

    @pl.when(cc < _NCHUNK)
    def _():
      pltpu.async_copy(h_hbm.at[sidx.at[cc]], rows.at[b], sems.at[b])

    @pl.when(cc > 0)
    def _():
      pc = cc - 1
      pb = 1 - b
      pltpu.make_async_copy(h_hbm.at[sidx.at[pc]], rows.at[pb],
                            sems.at[pb]).wait()
      pltpu.sync_copy(rows.at[pb], acc.at[didx.at[pc]], add=True)
    return carry
  lax.fori_loop(0, _NCHUNK + 1, chunk, 0)

  plsc.subcore_barrier()

  def outb(j, carry):
    r0 = sid * _RPT + j * _SR
    pltpu.sync_copy(acc.at[pl.ds(r0, _SR)], stg)
    pltpu.sync_copy(stg, out_hbm.at[cid, pl.ds(r0, _SR)])
    return carry
  lax.fori_loop(0, _RPT // _SR, outb, 0)


@functools.cache
def _get_segsum():
  mesh = plsc.VectorSubcoreMesh(
      core_axis_name="c", subcore_axis_name="s",
      num_cores=_NC, num_subcores=_NS)
  return pl.kernel(
      _segsum_body,
      out_type=jax.ShapeDtypeStruct((_NC, _HALF, _D), jnp.float32),
      mesh=mesh,
      scratch_types=[
          pltpu.VMEM((_NCHUNK, _CH), jnp.int32),
          pltpu.VMEM((_NCHUNK, _CH), jnp.int32),
          pltpu.VMEM((2, _CH, _D), jnp.float32),
          pltpu.VMEM((_SR, _D), jnp.float32),
          pltpu.VMEM_SHARED((_HALF + 8, _D), jnp.float32),
          pltpu.SemaphoreType.DMA((2,)),
      ],
  )


def _segsum(h, src, dst, zrows):
  return _get_segsum()(h, src, dst, zrows)


_BLK = 640
_NBLK = _NP // _BLK
_BPH = _HALF // _BLK


def _mlp_body(relu_out, xr, pr, War, bar, Wbr, bbr, outr):
  t = xr[...] + pr[0]
  u = jnp.maximum(
      jnp.dot(t, War[...], preferred_element_type=jnp.float32) + bar[...], 0.0)
  v = jnp.dot(u, Wbr[...], preferred_element_type=jnp.float32) + bbr[...]
  outr[...] = jnp.maximum(v, 0.0) if relu_out else v


def _mlp(x, parts, Wa, ba, Wb, bb, relu_out):
  return pl.pallas_call(
      functools.partial(_mlp_body, relu_out),
      grid=(_NBLK,),
      in_specs=[
          pl.BlockSpec((_BLK, _D), lambda i: (i, 0)),
          pl.BlockSpec((1, _BLK, _D), lambda i: (i // _BPH, i % _BPH, 0)),
          pl.BlockSpec((_D, _H), lambda i: (0, 0)),
          pl.BlockSpec((1, _H), lambda i: (0, 0)),
          pl.BlockSpec((_H, _H), lambda i: (0, 0)),
          pl.BlockSpec((1, _H), lambda i: (0, 0)),
      ],
      out_specs=pl.BlockSpec((_BLK, _H), lambda i: (i, 0)),
      out_shape=jax.ShapeDtypeStruct((_NP, _H), jnp.float32),
  )(x, parts, Wa, ba.reshape(1, _H), Wb, bb.reshape(1, _H))


def _final_body(xr, pr, War, bar, Wbr, bbr, batchr, linWr, linbr,
                outr, midr, sums, cnts):
  i = pl.program_id(0)

  @pl.when(i == 0)
  def _():
    sums[...] = jnp.zeros_like(sums)
    cnts[...] = jnp.zeros_like(cnts)

  t = xr[...] + pr[0]
  u = jnp.maximum(
      jnp.dot(t, War[...], preferred_element_type=jnp.float32) + bar[...], 0.0)
  v = jnp.dot(u, Wbr[...], preferred_element_type=jnp.float32) + bbr[...]
  onehot = (batchr[...] == lax.broadcasted_iota(jnp.int32, (_BLK, _G), 1)
            ).astype(jnp.float32)
  sums[...] += lax.dot_general(onehot, v, (((0,), (0,)), ((), ())),
                               preferred_element_type=jnp.float32)
  cnts[...] += lax.dot_general(onehot, jnp.ones((_BLK, 1), jnp.float32),
                               (((0,), (0,)), ((), ())),
                               preferred_element_type=jnp.float32)

  @pl.when(i == _NBLK - 1)
  def _():
    pooled = sums[...] / jnp.maximum(cnts[...], 1.0)
    midr[...] = pooled
    outr[...] = (jnp.dot(pooled, linWr[...], preferred_element_type=jnp.float32)
                 + linbr[...])


def _final(x, parts, Wa, ba, Wb, bb, batch2d, lin_W, lin_b):
  return pl.pallas_call(
      _final_body,
      grid=(_NBLK,),
      in_specs=[
          pl.BlockSpec((_BLK, _D), lambda i: (i, 0)),
          pl.BlockSpec((1, _BLK, _D), lambda i: (i // _BPH, i % _BPH, 0)),
          pl.BlockSpec((_D, _H), lambda i: (0, 0)),
          pl.BlockSpec((1, _H), lambda i: (0, 0)),
          pl.BlockSpec((_H, _H), lambda i: (0, 0)),
          pl.BlockSpec((1, _H), lambda i: (0, 0)),
          pl.BlockSpec((_BLK, 1), lambda i: (i, 0)),
          pl.BlockSpec((_H, _C), lambda i: (0, 0)),
          pl.BlockSpec((1, _C), lambda i: (0, 0)),
      ],
      out_specs=[
          pl.BlockSpec((_G, _C), lambda i: (0, 0)),
          pl.BlockSpec((_G, _H), lambda i: (0, 0)),
      ],
      out_shape=[
          jax.ShapeDtypeStruct((_G, _C), jnp.float32),
          jax.ShapeDtypeStruct((_G, _H), jnp.float32),
      ],
      scratch_shapes=[
          pltpu.VMEM((_G, _H), jnp.float32),
          pltpu.VMEM((_G, 1), jnp.float32),
      ],
  )(x, parts, Wa, ba.reshape(1, _H), Wb, bb.reshape(1, _H),
    batch2d, lin_W, lin_b.reshape(1, _C))


def kernel(x, edge_index, batch, W1a, b1a, W1b, b1b, W2a, b2a, W2b, b2b,
           W3a, b3a, W3b, b3b, lin_W, lin_b):
  pad_e = _NS * _EPT - _E
  src = jnp.pad(edge_index[0].reshape(_NS, _E // _NS),
                ((0, 0), (0, pad_e // _NS))).reshape(_NS, _NCHUNK, _CH)
  dst = jnp.pad(edge_index[1].reshape(_NS, _E // _NS),
                ((0, 0), (0, pad_e // _NS)),
                constant_values=_DPAD).reshape(_NS, _NCHUNK, _CH)
  zrows = jnp.zeros((_SR, _D), jnp.float32)
  xp = jnp.pad(x, ((0, _NP - _N), (0, 0)))
  batch2d = jnp.pad(batch, (0, _NP - _N), constant_values=_G).reshape(_NP, 1)

  p = _segsum(xp, src, dst, zrows)
  h = _mlp(xp, p, W1a, b1a, W1b, b1b, relu_out=True)
  p = _segsum(h, src, dst, zrows)
  h = _mlp(h, p, W2a, b2a, W2b, b2b, relu_out=True)
  p = _segsum(h, src, dst, zrows)
  out, x_mid = _final(h, p, W3a, b3a, W3b, b3b, batch2d, lin_W, lin_b)
  return (out, x_mid)
